# Initial kernel scaffold; baseline (speedup 1.0000x reference)
import jax
import jax.numpy as jnp
from jax.experimental import pallas as pl


def kernel(sentence, emb_weight):
    raise NotImplementedError("write your pallas kernel here")


# speedup vs baseline: 5.0093x; 5.0093x over previous
import functools

import jax
import jax.numpy as jnp
from jax import lax
from jax.experimental import pallas as pl
from jax.experimental.pallas import tpu as pltpu
from jax.experimental.pallas import tpu_sc as plsc

VOCAB = 100000
DIM = 128
SEQ = 200
BATCH = 1024

NUM_WORKERS = 32
ROWS = SEQ * BATCH
ROWS_PER_WORKER = ROWS // NUM_WORKERS
CHUNK = 128
CHUNKS_PER_WORKER = ROWS_PER_WORKER // CHUNK


def _make_sc_gather():
    mesh = plsc.VectorSubcoreMesh(core_axis_name="c", subcore_axis_name="s")

    @functools.partial(
        pl.kernel,
        mesh=mesh,
        out_type=jax.ShapeDtypeStruct((ROWS, DIM), jnp.float32),
        scratch_types=[
            pltpu.VMEM((CHUNK,), jnp.int32),
            pltpu.VMEM((CHUNK, DIM), jnp.float32),
            pltpu.SemaphoreType.DMA,
        ],
    )
    def gather_kernel(idx_hbm, table_hbm, out_hbm, idx_v, rows_v, sem):
        wid = lax.axis_index("s") * 2 + lax.axis_index("c")
        base = wid * ROWS_PER_WORKER

        def chunk_body(i, carry):
            off = base + i * CHUNK
            pltpu.sync_copy(idx_hbm.at[pl.ds(off, CHUNK)], idx_v)
            pltpu.async_copy(table_hbm.at[idx_v], rows_v, sem).wait()
            pltpu.sync_copy(rows_v, out_hbm.at[pl.ds(off, CHUNK)])
            return carry

        lax.fori_loop(0, CHUNKS_PER_WORKER, chunk_body, 0)

    return gather_kernel


_sc_gather = _make_sc_gather()


@jax.jit
def kernel(sentence, emb_weight):
    idx = jnp.transpose(sentence).reshape(ROWS)
    out = _sc_gather(idx, emb_weight)
    return out.reshape(BATCH, SEQ, DIM)


# speedup vs baseline: 7.5384x; 1.5049x over previous
import functools

import jax
import jax.numpy as jnp
from jax import lax
from jax.experimental import pallas as pl
from jax.experimental.pallas import tpu as pltpu
from jax.experimental.pallas import tpu_sc as plsc

VOCAB = 100000
DIM = 128
SEQ = 200
BATCH = 1024

NUM_WORKERS = 32
ROWS = SEQ * BATCH
ROWS_PER_WORKER = ROWS // NUM_WORKERS
CHUNK = 128
CHUNKS_PER_WORKER = ROWS_PER_WORKER // CHUNK


def _make_sc_gather():
    mesh = plsc.VectorSubcoreMesh(core_axis_name="c", subcore_axis_name="s")
    nsteps = CHUNKS_PER_WORKER // 2

    @functools.partial(
        pl.kernel,
        mesh=mesh,
        out_type=jax.ShapeDtypeStruct((ROWS, DIM), jnp.float32),
        scratch_types=[
            pltpu.VMEM((CHUNK,), jnp.int32),
            pltpu.VMEM((CHUNK,), jnp.int32),
            pltpu.VMEM((CHUNK, DIM), jnp.float32),
            pltpu.VMEM((CHUNK, DIM), jnp.float32),
            pltpu.SemaphoreType.DMA,
            pltpu.SemaphoreType.DMA,
            pltpu.SemaphoreType.DMA,
            pltpu.SemaphoreType.DMA,
        ],
    )
    def gather_kernel(idx_hbm, table_hbm, out_hbm,
                      idx0, idx1, rows0, rows1, g0, g1, w0, w1):
        wid = lax.axis_index("s") * 2 + lax.axis_index("c")
        base = wid * ROWS_PER_WORKER
        idx_v = (idx0, idx1)
        rows_v = (rows0, rows1)
        gsem = (g0, g1)
        wsem = (w0, w1)

        def step(g, carry):
            for j in range(2):
                off = base + (2 * g + j) * CHUNK
                @pl.when(g > 0)
                def _():
                    pltpu.make_async_copy(
                        rows_v[j], out_hbm.at[pl.ds(off, CHUNK)], wsem[j]
                    ).wait()
                pltpu.sync_copy(idx_hbm.at[pl.ds(off, CHUNK)], idx_v[j])
                pltpu.async_copy(table_hbm.at[idx_v[j]], rows_v[j], gsem[j])
            for j in range(2):
                off = base + (2 * g + j) * CHUNK
                pltpu.make_async_copy(
                    table_hbm.at[idx_v[j]], rows_v[j], gsem[j]
                ).wait()
                pltpu.async_copy(rows_v[j], out_hbm.at[pl.ds(off, CHUNK)], wsem[j])
            return carry

        lax.fori_loop(0, nsteps, step, 0)
        for j in range(2):
            off = base + (2 * (nsteps - 1) + j) * CHUNK
            pltpu.make_async_copy(
                rows_v[j], out_hbm.at[pl.ds(off, CHUNK)], wsem[j]
            ).wait()

    return gather_kernel


_sc_gather = _make_sc_gather()


@jax.jit
def kernel(sentence, emb_weight):
    idx = jnp.transpose(sentence).reshape(ROWS)
    out = _sc_gather(idx, emb_weight)
    return out.reshape(BATCH, SEQ, DIM)


# speedup vs baseline: 8.4717x; 1.1238x over previous
import functools

import jax
import jax.numpy as jnp
from jax import lax
from jax.experimental import pallas as pl
from jax.experimental.pallas import tpu as pltpu
from jax.experimental.pallas import tpu_sc as plsc

VOCAB = 100000
DIM = 128
SEQ = 200
BATCH = 1024

NUM_WORKERS = 32
ROWS = SEQ * BATCH
ROWS_PER_WORKER = ROWS // NUM_WORKERS
CHUNK = 128
CHUNKS_PER_WORKER = ROWS_PER_WORKER // CHUNK


NBUF = 4


def _make_sc_gather():
    mesh = plsc.VectorSubcoreMesh(core_axis_name="c", subcore_axis_name="s")
    nsteps = CHUNKS_PER_WORKER // NBUF

    @functools.partial(
        pl.kernel,
        mesh=mesh,
        out_type=jax.ShapeDtypeStruct((ROWS, DIM), jnp.float32),
        scratch_types=[
            tuple(pltpu.VMEM((CHUNK,), jnp.int32) for _ in range(NBUF)),
            tuple(pltpu.VMEM((CHUNK, DIM), jnp.float32) for _ in range(NBUF)),
            tuple(pltpu.SemaphoreType.DMA for _ in range(NBUF)),
            tuple(pltpu.SemaphoreType.DMA for _ in range(NBUF)),
            tuple(pltpu.SemaphoreType.DMA for _ in range(NBUF)),
        ],
    )
    def gather_kernel(idx_hbm, table_hbm, out_hbm,
                      idx_v, rows_v, isem, gsem, wsem):
        wid = lax.axis_index("s") * 2 + lax.axis_index("c")
        base = wid * ROWS_PER_WORKER

        for j in range(NBUF):
            pltpu.async_copy(
                idx_hbm.at[pl.ds(base + j * CHUNK, CHUNK)], idx_v[j], isem[j]
            )

        def step(g, carry):
            for j in range(NBUF):
                c = NBUF * g + j
                off = base + c * CHUNK
                @pl.when(g > 0)
                def _():
                    pltpu.make_async_copy(
                        rows_v[j], out_hbm.at[pl.ds(off, CHUNK)], wsem[j]
                    ).wait()
                pltpu.make_async_copy(
                    idx_hbm.at[pl.ds(off, CHUNK)], idx_v[j], isem[j]
                ).wait()
                pltpu.async_copy(table_hbm.at[idx_v[j]], rows_v[j], gsem[j])
            for j in range(NBUF):
                c = NBUF * g + j
                off = base + c * CHUNK
                pltpu.make_async_copy(
                    table_hbm.at[idx_v[j]], rows_v[j], gsem[j]
                ).wait()
                pltpu.async_copy(rows_v[j], out_hbm.at[pl.ds(off, CHUNK)], wsem[j])
                @pl.when(g < nsteps - 1)
                def _():
                    pltpu.async_copy(
                        idx_hbm.at[pl.ds(off + NBUF * CHUNK, CHUNK)],
                        idx_v[j], isem[j],
                    )
            return carry

        lax.fori_loop(0, nsteps, step, 0)
        for j in range(NBUF):
            off = base + (NBUF * (nsteps - 1) + j) * CHUNK
            pltpu.make_async_copy(
                rows_v[j], out_hbm.at[pl.ds(off, CHUNK)], wsem[j]
            ).wait()

    return gather_kernel


_sc_gather = _make_sc_gather()


@jax.jit
def kernel(sentence, emb_weight):
    idx = jnp.transpose(sentence).reshape(ROWS)
    out = _sc_gather(idx, emb_weight)
    return out.reshape(BATCH, SEQ, DIM)
